# baseline (speedup 1.0000x reference)
import jax
import jax.numpy as jnp
from jax.experimental import pallas as pl
from jax.experimental.pallas import tpu as pltpu

NKPTS = 8
NFRM = 2
NNODES = NKPTS * NFRM
C0 = 512
BATCH = 1024
FEAT = 2048
TB = 128
TC = 512
NB = BATCH // TB
NC = (NNODES * C0) // TC

_F32 = jnp.float32
_BF16 = jnp.bfloat16


def _elu(v):
    return jnp.where(v > 0, v, jnp.exp(v) - 1.0)


def _spiral_layer(nodes, Wv, bb, cin, act):
    Ws = [Wv[s * cin:(s + 1) * cin, :] for s in range(9)]

    def dot(a, w):
        return jnp.dot(a, w, preferred_element_type=_F32)

    T = [dot(nodes[m], Ws[8]) for m in range(NNODES)]

    new_nodes = [None] * NNODES
    for f in range(NFRM):
        base = f * NKPTS
        A = {j: dot(nodes[base + j], Ws[j]) for j in range(1, NKPTS)}
        B = {j: dot(nodes[base + j], Ws[j + 1]) for j in range(NKPTS - 1)}
        C = [None]
        acc = None
        for j in range(NKPTS - 1):
            acc = B[j] if acc is None else acc + B[j]
            C.append(acc)
        D = [None] * NKPTS
        acc = None
        for n in range(NKPTS - 2, -1, -1):
            acc = A[n + 1] if acc is None else acc + A[n + 1]
            D[n] = acc
        for n in range(NKPTS):
            mate = (1 - f) * NKPTS + n
            val = dot(nodes[base + n], Ws[0]) + T[mate] + bb
            if C[n] is not None:
                val = val + C[n]
            if D[n] is not None:
                val = val + D[n]
            new_nodes[base + n] = _elu(val) if act else val
    return new_nodes


def _fused_kernel(x_ref, w0_ref, b0_ref, w1_ref, b1_ref, w2_ref, b2_ref,
                  w3_ref, b3_ref, w4e_ref, b4_ref, o_ref, h_ref):
    c = pl.program_id(0)
    b = pl.program_id(1)

    xb = x_ref[pl.ds(b * TB, TB), :]
    acc = jnp.dot(xb, w0_ref[...], preferred_element_type=_F32)
    acc = acc + b0_ref[:, pl.ds(c * TC, TC)]
    h_ref[pl.ds(b * TB, TB), pl.ds(c * TC, TC)] = acc.astype(_BF16)

    @pl.when(c == NC - 1)
    def _stack():
        hb = h_ref[pl.ds(b * TB, TB), :].astype(_F32)
        nodes = [hb[:, n * C0:(n + 1) * C0] for n in range(NNODES)]
        nodes = _spiral_layer(nodes, w1_ref[...], b1_ref[...], 512, act=True)
        nodes = _spiral_layer(nodes, w2_ref[...], b2_ref[...], 512, act=True)
        nodes = _spiral_layer(nodes, w3_ref[...], b3_ref[...], 256, act=True)
        hcat = jnp.concatenate(nodes, axis=1)
        o_ref[...] = (
            jnp.dot(hcat, w4e_ref[...], preferred_element_type=_F32)
            + b4_ref[...]
        )


def _expand_w4(W4):
    cin = 128
    zblk = jnp.zeros((cin, 3), W4.dtype)
    cols = []
    for n in range(NNODES):
        f, r = divmod(n, NKPTS)
        base = f * NKPTS
        rows = []
        for m in range(NNODES):
            if m == n:
                s = 0
            elif base <= m < base + NKPTS:
                j = m - base
                s = j + 1 if j < r else j
            elif m == (1 - f) * NKPTS + r:
                s = 8
            else:
                s = None
            rows.append(zblk if s is None else W4[s * cin:(s + 1) * cin, :])
        cols.append(jnp.concatenate(rows, axis=0))
    return jnp.concatenate(cols, axis=1)


def kernel(x, W0, b0, W1, b1, W2, b2, W3, b3, W4, b4):
    W4e = _expand_w4(W4)
    const2 = lambda c, b: (0, 0)

    out = pl.pallas_call(
        _fused_kernel,
        grid=(NC, NB),
        in_specs=[
            pl.BlockSpec((BATCH, FEAT), const2),
            pl.BlockSpec((FEAT, TC), lambda c, b: (0, c)),
            pl.BlockSpec((1, NNODES * C0), const2),
            pl.BlockSpec(W1.shape, const2),
            pl.BlockSpec((1, 512), const2),
            pl.BlockSpec(W2.shape, const2),
            pl.BlockSpec((1, 256), const2),
            pl.BlockSpec(W3.shape, const2),
            pl.BlockSpec((1, 128), const2),
            pl.BlockSpec((NNODES * 128, NNODES * 3), const2),
            pl.BlockSpec((1, NNODES * 3), const2),
        ],
        out_specs=pl.BlockSpec((TB, NNODES * 3), lambda c, b: (b, 0)),
        out_shape=jax.ShapeDtypeStruct((BATCH, NNODES * 3), _F32),
        scratch_shapes=[pltpu.VMEM((BATCH, NNODES * C0), _BF16)],
        compiler_params=pltpu.CompilerParams(
            vmem_limit_bytes=63 * 1024 * 1024,
        ),
    )(x, W0, b0.reshape(1, -1), W1, b1.reshape(1, -1), W2,
      b2.reshape(1, -1), W3, b3.reshape(1, -1), W4e,
      jnp.tile(b4, NNODES).reshape(1, -1))

    return out.reshape(BATCH, NNODES, 3)


# speedup vs baseline: 1.4611x; 1.4611x over previous
import jax
import jax.numpy as jnp
from jax.experimental import pallas as pl

NKPTS = 8
NFRM = 2
NNODES = NKPTS * NFRM
C0 = 512
BATCH = 1024
FEAT = 2048

_F32 = jnp.float32
_BF16 = jnp.bfloat16


def _elu(v):
    return jnp.where(v > 0, v, jnp.exp(v) - 1.0)


def _dense_kernel(x_ref, w_ref, b_ref, o_ref):
    acc = jnp.dot(x_ref[...].astype(_BF16), w_ref[...].astype(_BF16),
                  preferred_element_type=_F32)
    o_ref[...] = (acc + b_ref[...]).astype(_BF16)


def _spiral_layer(nodes, Wv, bb, cin, act):
    Ws = [Wv[s * cin:(s + 1) * cin, :] for s in range(9)]

    def dot(a, w):
        return jnp.dot(a, w, preferred_element_type=_F32)

    T = [dot(nodes[m], Ws[8]) for m in range(NNODES)]

    new_nodes = [None] * NNODES
    for f in range(NFRM):
        base = f * NKPTS
        A = {j: dot(nodes[base + j], Ws[j]) for j in range(1, NKPTS)}
        B = {j: dot(nodes[base + j], Ws[j + 1]) for j in range(NKPTS - 1)}
        C = [None]
        acc = None
        for j in range(NKPTS - 1):
            acc = B[j] if acc is None else acc + B[j]
            C.append(acc)
        D = [None] * NKPTS
        acc = None
        for n in range(NKPTS - 2, -1, -1):
            acc = A[n + 1] if acc is None else acc + A[n + 1]
            D[n] = acc
        for n in range(NKPTS):
            mate = (1 - f) * NKPTS + n
            val = dot(nodes[base + n], Ws[0]) + T[mate] + bb
            if C[n] is not None:
                val = val + C[n]
            if D[n] is not None:
                val = val + D[n]
            new_nodes[base + n] = (_elu(val) if act else val).astype(_BF16)
    return new_nodes


def _stack_kernel(h_ref, w1_ref, b1_ref, w2_ref, b2_ref, w3_ref, b3_ref,
                  w4e_ref, b4_ref, o_ref):
    nodes = [h_ref[:, n * C0:(n + 1) * C0] for n in range(NNODES)]
    nodes = _spiral_layer(nodes, w1_ref[...].astype(_BF16), b1_ref[...], 512, act=True)
    nodes = _spiral_layer(nodes, w2_ref[...].astype(_BF16), b2_ref[...], 512, act=True)
    nodes = _spiral_layer(nodes, w3_ref[...].astype(_BF16), b3_ref[...], 256, act=True)
    hcat = jnp.concatenate(nodes, axis=1)
    o_ref[...] = (
        jnp.dot(hcat, w4e_ref[...].astype(_BF16), preferred_element_type=_F32)
        + b4_ref[...]
    )


def _expand_w4(W4):
    cin = 128
    zblk = jnp.zeros((cin, 3), W4.dtype)
    cols = []
    for n in range(NNODES):
        f, r = divmod(n, NKPTS)
        base = f * NKPTS
        rows = []
        for m in range(NNODES):
            if m == n:
                s = 0
            elif base <= m < base + NKPTS:
                j = m - base
                s = j + 1 if j < r else j
            elif m == (1 - f) * NKPTS + r:
                s = 8
            else:
                s = None
            rows.append(zblk if s is None else W4[s * cin:(s + 1) * cin, :])
        cols.append(jnp.concatenate(rows, axis=0))
    return jnp.concatenate(cols, axis=1)


def kernel(x, W0, b0, W1, b1, W2, b2, W3, b3, W4, b4):
    TB = 256
    TC = 1024
    nb = BATCH // TB
    nc = (NNODES * C0) // TC

    h = pl.pallas_call(
        _dense_kernel,
        grid=(nc,),
        in_specs=[
            pl.BlockSpec((BATCH, FEAT), lambda c: (0, 0)),
            pl.BlockSpec((FEAT, TC), lambda c: (0, c)),
            pl.BlockSpec((1, TC), lambda c: (0, c)),
        ],
        out_specs=pl.BlockSpec((BATCH, TC), lambda c: (0, c)),
        out_shape=jax.ShapeDtypeStruct((BATCH, NNODES * C0), _BF16),
    )(x, W0, b0.reshape(1, -1))

    W4e = _expand_w4(W4)

    const = lambda b: (0, 0)
    out = pl.pallas_call(
        _stack_kernel,
        grid=(nb,),
        in_specs=[
            pl.BlockSpec((TB, NNODES * C0), lambda b: (b, 0)),
            pl.BlockSpec(W1.shape, const),
            pl.BlockSpec((1, 512), const),
            pl.BlockSpec(W2.shape, const),
            pl.BlockSpec((1, 256), const),
            pl.BlockSpec(W3.shape, const),
            pl.BlockSpec((1, 128), const),
            pl.BlockSpec((NNODES * 128, NNODES * 3), const),
            pl.BlockSpec((1, NNODES * 3), const),
        ],
        out_specs=pl.BlockSpec((TB, NNODES * 3), lambda b: (b, 0)),
        out_shape=jax.ShapeDtypeStruct((BATCH, NNODES * 3), _F32),
    )(h, W1, b1.reshape(1, -1), W2, b2.reshape(1, -1), W3,
      b3.reshape(1, -1), W4e, jnp.tile(b4, NNODES).reshape(1, -1))

    return out.reshape(BATCH, NNODES, 3)
